# baseline (speedup 1.0000x reference)
import jax
import jax.numpy as jnp
from jax import lax
from jax.experimental import pallas as pl
from jax.experimental.pallas import tpu as pltpu
from jax.experimental.pallas import tpu_sc as plsc

B = 4096
EMB = 64
T_LEN = 20
R_LEN = 200
H1 = 256
H2 = 128
NCLS = 2

NUM_CORES = 2
NUM_SUBCORES = 16
NW = NUM_CORES * NUM_SUBCORES
BPW = B // NW

TG = 4
TROWS = TG * T_LEN
NTG = BPW // TG
RC0 = 128
RC1 = R_LEN - RC0


def _bow_body(tg_hbm, rg_hbm, th_hbm, rh_hbm, emb_hbm, out_hbm,
              tg_v, rg_v, th_v, rh_v,
              c0A, c1A, c0B, c1B, acc_v,
              semRA, semRB):
    wid = lax.axis_index("s") * NUM_CORES + lax.axis_index("c")
    base = wid * BPW

    pltpu.sync_copy(tg_hbm.at[pl.ds(base * T_LEN, BPW * T_LEN)], tg_v.at[pl.ds(0, BPW * T_LEN)])
    pltpu.sync_copy(rg_hbm.at[pl.ds(base * R_LEN, BPW * R_LEN)], rg_v.at[pl.ds(0, BPW * R_LEN)])
    pltpu.sync_copy(th_hbm.at[pl.ds(base * (T_LEN // 4), BPW * (T_LEN // 4))], th_v.at[pl.ds(0, BPW * (T_LEN // 4))])
    pltpu.sync_copy(rh_hbm.at[pl.ds(base * (R_LEN // 4), BPW * (R_LEN // 4))], rh_v.at[pl.ds(0, BPW * (R_LEN // 4))])

    def fire_title(g, c0, sem):
        pltpu.async_copy(emb_hbm.at[tg_v.at[pl.ds(g * TROWS, TROWS)]], c0.at[pl.ds(0, TROWS)], sem)

    def drain_title(c0, sem):
        pltpu.make_async_copy(emb_hbm.at[pl.ds(0, TROWS)], c0.at[pl.ds(0, TROWS)], sem).wait()

    def fire_review(r, c0, c1, sem):
        pltpu.async_copy(emb_hbm.at[rg_v.at[pl.ds(r * R_LEN, RC0)]], c0, sem)
        pltpu.async_copy(emb_hbm.at[rg_v.at[pl.ds(r * R_LEN + RC0, RC1)]], c1, sem)

    def drain_review(c0, c1, sem):
        pltpu.make_async_copy(emb_hbm.at[pl.ds(0, RC0)], c0, sem).wait()
        pltpu.make_async_copy(emb_hbm.at[pl.ds(0, RC1)], c1, sem).wait()

    zero4 = (jnp.zeros((16,), jnp.float32),) * 4

    def hsel(wv, pos):
        w = wv[pos // 4]
        sh = 8 * (pos % 4)
        h = w if sh == 0 else jnp.right_shift(w, sh)
        return jnp.bitwise_and(h, 127)

    def process_title(g, tbuf):
        wv0 = th_v[pl.ds(g * (TROWS // 4), 16)]
        wv1 = th_v[pl.ds(g * (TROWS // 4) + 16, 16)]
        for k in range(TG):
            accs = zero4
            for l in range(T_LEN):
                row = T_LEN * k + l
                h = hsel(wv0, row) if row < 64 else hsel(wv1, row - 64)
                accs = tuple(a + tbuf[row, pl.ds(h + 16 * c, 16)]
                             for c, a in enumerate(accs))
            bag = TG * g + k
            for c in range(4):
                acc_v[bag, pl.ds(16 * c, 16)] = accs[c]

    def process_review(r, c0, c1):
        def grp(j3, accs):
            wv = rh_v[pl.ds(r * (R_LEN // 4) + j3 * 16, 16)]
            for l in range(64):
                row = j3 * 64 + l
                h = hsel(wv, l)
                accs = tuple(a + c0[row, pl.ds(h + 16 * c, 16)]
                             for c, a in enumerate(accs))
            return accs

        accs = lax.fori_loop(0, 2, grp, zero4)
        wv = rh_v[pl.ds(r * (R_LEN // 4) + 32, 16)]
        for l in range(64):
            h = hsel(wv, l)
            accs = tuple(a + c1[l, pl.ds(h + 16 * c, 16)]
                         for c, a in enumerate(accs))
        wv = rh_v[pl.ds(r * (R_LEN // 4) + 48, 16)]
        for l in range(R_LEN - 192):
            h = hsel(wv, l)
            accs = tuple(a + c1[64 + l, pl.ds(h + 16 * c, 16)]
                         for c, a in enumerate(accs))
        for c in range(4):
            acc_v[r, pl.ds(EMB + 16 * c, 16)] = accs[c]

    fire_title(0, c0A, semRA)
    fire_title(1, c0B, semRB)

    def tloop(j, carry):
        g0 = 2 * j
        drain_title(c0A, semRA)
        process_title(g0, c0A)

        @pl.when(j < NTG // 2 - 1)
        def _():
            fire_title(g0 + 2, c0A, semRA)

        drain_title(c0B, semRB)
        process_title(g0 + 1, c0B)

        @pl.when(j < NTG // 2 - 1)
        def _():
            fire_title(g0 + 3, c0B, semRB)

        return carry

    lax.fori_loop(0, NTG // 2, tloop, 0)

    fire_review(0, c0A, c1A, semRA)
    fire_review(1, c0B, c1B, semRB)

    def rloop(j, carry):
        r0 = 2 * j
        drain_review(c0A, c1A, semRA)
        process_review(r0, c0A, c1A)

        @pl.when(j < BPW // 2 - 1)
        def _():
            fire_review(r0 + 2, c0A, c1A, semRA)

        drain_review(c0B, c1B, semRB)
        process_review(r0 + 1, c0B, c1B)

        @pl.when(j < BPW // 2 - 1)
        def _():
            fire_review(r0 + 3, c0B, c1B, semRB)

        return carry

    lax.fori_loop(0, BPW // 2, rloop, 0)

    pltpu.sync_copy(acc_v, out_hbm.at[pl.ds(base, BPW)])


@jax.jit
def _bow(tg, rg, th8, rh8, emb2):
    mesh = plsc.VectorSubcoreMesh(core_axis_name="c", subcore_axis_name="s")
    return pl.kernel(
        _bow_body,
        out_type=jax.ShapeDtypeStruct((B, 2 * EMB), jnp.float32),
        mesh=mesh,
        scratch_types=[
            pltpu.VMEM((BPW * T_LEN,), jnp.int32),
            pltpu.VMEM((BPW * R_LEN + 8,), jnp.int32),
            pltpu.VMEM((BPW * (T_LEN // 4) + 16,), jnp.int32),
            pltpu.VMEM((BPW * (R_LEN // 4) + 16,), jnp.int32),
            pltpu.VMEM((RC0, 2 * EMB), jnp.float32),
            pltpu.VMEM((RC1, 2 * EMB), jnp.float32),
            pltpu.VMEM((RC0, 2 * EMB), jnp.float32),
            pltpu.VMEM((RC1, 2 * EMB), jnp.float32),
            pltpu.VMEM((BPW, 2 * EMB), jnp.float32),
            pltpu.SemaphoreType.DMA,
            pltpu.SemaphoreType.DMA,
        ],
    )(tg, rg, th8, rh8, emb2)


TRB = 32768
TRH = TRB // 2


def _tr_body(x_ref, o_ref):
    x = x_ref[...]
    for s in range(0, TRH, 512):
        o_ref[s:s + 512, 0:64] = jnp.transpose(x[:, s:s + 512])
        o_ref[s:s + 512, 64:128] = jnp.transpose(x[:, TRH + s:TRH + s + 512])


def _transpose(embT):
    nblk = (1000000 + TRB - 1) // TRB
    return pl.pallas_call(
        _tr_body,
        grid=(nblk,),
        in_specs=[pl.BlockSpec((64, TRB), lambda j: (0, j))],
        out_specs=pl.BlockSpec((TRH, 128), lambda j: (j, 0)),
        out_shape=jax.ShapeDtypeStruct((nblk * TRH, 128), jnp.float32),
    )(embT)


def _mlp_body(x_ref, w1_ref, b1_ref, w2_ref, b2_ref, w3_ref, b3_ref, o_ref):
    x = x_ref[...]
    h = jnp.dot(x, w1_ref[...], preferred_element_type=jnp.float32) + b1_ref[...]
    h = jnp.maximum(h, 0.0)
    h = jnp.dot(h, w2_ref[...], preferred_element_type=jnp.float32) + b2_ref[...]
    h = jnp.maximum(h, 0.0)
    o_ref[...] = jnp.dot(h, w3_ref[...], preferred_element_type=jnp.float32) + b3_ref[...]


def _mlp(x, w1t, b1r, w2t, b2r, w3t, b3r):
    BLK = 512
    return pl.pallas_call(
        _mlp_body,
        grid=(B // BLK,),
        in_specs=[
            pl.BlockSpec((BLK, 2 * EMB), lambda i: (i, 0)),
            pl.BlockSpec((2 * EMB, H1), lambda i: (0, 0)),
            pl.BlockSpec((1, H1), lambda i: (0, 0)),
            pl.BlockSpec((H1, H2), lambda i: (0, 0)),
            pl.BlockSpec((1, H2), lambda i: (0, 0)),
            pl.BlockSpec((H2, NCLS), lambda i: (0, 0)),
            pl.BlockSpec((1, NCLS), lambda i: (0, 0)),
        ],
        out_specs=pl.BlockSpec((BLK, NCLS), lambda i: (i, 0)),
        out_shape=jax.ShapeDtypeStruct((B, NCLS), jnp.float32),
    )(x, w1t, b1r, w2t, b2r, w3t, b3r)


def kernel(title_inputs, review_inputs, emb, W1, b1, W2, b2, W3, b3):
    tf = title_inputs.reshape(-1)
    rf = review_inputs.reshape(-1)
    tblk = jnp.right_shift(tf, 15)
    tloc = jnp.bitwise_and(tf, TRB - 1)
    rblk = jnp.right_shift(rf, 15)
    rloc = jnp.bitwise_and(rf, TRB - 1)
    tg = tblk * TRH + jnp.bitwise_and(tloc, TRH - 1)
    rg = rblk * TRH + jnp.bitwise_and(rloc, TRH - 1)
    th8 = jnp.left_shift(jnp.right_shift(tloc, 14), 6).astype(jnp.int8)
    rh8 = jnp.left_shift(jnp.right_shift(rloc, 14), 6).astype(jnp.int8)
    th32 = lax.bitcast_convert_type(th8.reshape(-1, 4), jnp.int32)
    rh32 = lax.bitcast_convert_type(rh8.reshape(-1, 4), jnp.int32)
    emb2 = _transpose(emb.T)
    combined = _bow(tg, rg, th32, rh32, emb2)
    return _mlp(combined, W1.T, b1.reshape(1, -1),
                W2.T, b2.reshape(1, -1), W3.T, b3.reshape(1, -1))


# speedup vs baseline: 1.0149x; 1.0149x over previous
import jax
import jax.numpy as jnp
from jax import lax
from jax.experimental import pallas as pl
from jax.experimental.pallas import tpu as pltpu
from jax.experimental.pallas import tpu_sc as plsc

B = 4096
EMB = 64
T_LEN = 20
R_LEN = 200
H1 = 256
H2 = 128
NCLS = 2

NUM_CORES = 2
NUM_SUBCORES = 16
NW = NUM_CORES * NUM_SUBCORES
BPW = B // NW

TG = 4
TROWS = TG * T_LEN
NTG = BPW // TG
RC0 = 128
RC1 = R_LEN - RC0


def _bow_body(tg_hbm, rg_hbm, th_hbm, rh_hbm, emb_hbm, out_hbm,
              tg_v, rg_v, th_v, rh_v,
              c0A, c1A, c0B, c1B, acc_v,
              semRA, semRB):
    wid = lax.axis_index("s") * NUM_CORES + lax.axis_index("c")
    base = wid * BPW

    pltpu.sync_copy(tg_hbm.at[pl.ds(base * T_LEN, BPW * T_LEN)], tg_v.at[pl.ds(0, BPW * T_LEN)])
    pltpu.sync_copy(rg_hbm.at[pl.ds(base * R_LEN, BPW * R_LEN)], rg_v.at[pl.ds(0, BPW * R_LEN)])
    pltpu.sync_copy(th_hbm.at[pl.ds(base * (T_LEN // 4), BPW * (T_LEN // 4))], th_v.at[pl.ds(0, BPW * (T_LEN // 4))])
    pltpu.sync_copy(rh_hbm.at[pl.ds(base * (R_LEN // 4), BPW * (R_LEN // 4))], rh_v.at[pl.ds(0, BPW * (R_LEN // 4))])

    def fire_title(g, c0, sem):
        pltpu.async_copy(emb_hbm.at[tg_v.at[pl.ds(g * TROWS, TROWS)]], c0.at[pl.ds(0, TROWS)], sem)

    def drain_title(c0, sem):
        pltpu.make_async_copy(emb_hbm.at[pl.ds(0, TROWS)], c0.at[pl.ds(0, TROWS)], sem).wait()

    def fire_c0(r, c0, sem):
        pltpu.async_copy(emb_hbm.at[rg_v.at[pl.ds(r * R_LEN, RC0)]], c0, sem)

    def fire_c1(r, c1, sem):
        pltpu.async_copy(emb_hbm.at[rg_v.at[pl.ds(r * R_LEN + RC0, RC1)]], c1, sem)

    def fire_review(r, c0, c1, sem):
        fire_c0(r, c0, sem)
        fire_c1(r, c1, sem)

    def drain_review(c0, c1, sem):
        pltpu.make_async_copy(emb_hbm.at[pl.ds(0, RC0)], c0, sem).wait()
        pltpu.make_async_copy(emb_hbm.at[pl.ds(0, RC1)], c1, sem).wait()

    zero4 = (jnp.zeros((16,), jnp.float32),) * 4

    def hsel(wv, pos):
        w = wv[pos // 4]
        sh = 8 * (pos % 4)
        h = w if sh == 0 else jnp.right_shift(w, sh)
        return jnp.bitwise_and(h, 127)

    def process_title(g, tbuf):
        wv0 = th_v[pl.ds(g * (TROWS // 4), 16)]
        wv1 = th_v[pl.ds(g * (TROWS // 4) + 16, 16)]
        for k in range(TG):
            accs = zero4
            for l in range(T_LEN):
                row = T_LEN * k + l
                h = hsel(wv0, row) if row < 64 else hsel(wv1, row - 64)
                accs = tuple(a + tbuf[row, pl.ds(h + 16 * c, 16)]
                             for c, a in enumerate(accs))
            bag = TG * g + k
            for c in range(4):
                acc_v[bag, pl.ds(16 * c, 16)] = accs[c]

    def process_c0(r, c0):
        def grp(j3, accs):
            wv = rh_v[pl.ds(r * (R_LEN // 4) + j3 * 16, 16)]
            for l in range(64):
                row = j3 * 64 + l
                h = hsel(wv, l)
                accs = tuple(a + c0[row, pl.ds(h + 16 * c, 16)]
                             for c, a in enumerate(accs))
            return accs

        return lax.fori_loop(0, 2, grp, zero4)

    def process_c1(r, c1, accs):
        wv = rh_v[pl.ds(r * (R_LEN // 4) + 32, 16)]
        for l in range(64):
            h = hsel(wv, l)
            accs = tuple(a + c1[l, pl.ds(h + 16 * c, 16)]
                         for c, a in enumerate(accs))
        wv = rh_v[pl.ds(r * (R_LEN // 4) + 48, 16)]
        for l in range(R_LEN - 192):
            h = hsel(wv, l)
            accs = tuple(a + c1[64 + l, pl.ds(h + 16 * c, 16)]
                         for c, a in enumerate(accs))
        for c in range(4):
            acc_v[r, pl.ds(EMB + 16 * c, 16)] = accs[c]

    fire_title(0, c0A, semRA)
    fire_title(1, c0B, semRB)

    def tloop(j, carry):
        g0 = 2 * j
        drain_title(c0A, semRA)
        process_title(g0, c0A)

        @pl.when(j < NTG // 2 - 1)
        def _():
            fire_title(g0 + 2, c0A, semRA)

        drain_title(c0B, semRB)
        process_title(g0 + 1, c0B)

        @pl.when(j < NTG // 2 - 1)
        def _():
            fire_title(g0 + 3, c0B, semRB)

        return carry

    lax.fori_loop(0, NTG // 2, tloop, 0)

    fire_review(0, c0A, c1A, semRA)
    fire_review(1, c0B, c1B, semRB)

    def rloop(j, carry):
        r0 = 2 * j
        drain_review(c0A, c1A, semRA)
        accsA = process_c0(r0, c0A)

        @pl.when(j < BPW // 2 - 1)
        def _():
            fire_c0(r0 + 2, c0A, semRA)

        process_c1(r0, c1A, accsA)

        @pl.when(j < BPW // 2 - 1)
        def _():
            fire_c1(r0 + 2, c1A, semRA)

        drain_review(c0B, c1B, semRB)
        accsB = process_c0(r0 + 1, c0B)

        @pl.when(j < BPW // 2 - 1)
        def _():
            fire_c0(r0 + 3, c0B, semRB)

        process_c1(r0 + 1, c1B, accsB)

        @pl.when(j < BPW // 2 - 1)
        def _():
            fire_c1(r0 + 3, c1B, semRB)

        return carry

    lax.fori_loop(0, BPW // 2, rloop, 0)

    pltpu.sync_copy(acc_v, out_hbm.at[pl.ds(base, BPW)])


@jax.jit
def _bow(tg, rg, th8, rh8, emb2):
    mesh = plsc.VectorSubcoreMesh(core_axis_name="c", subcore_axis_name="s")
    return pl.kernel(
        _bow_body,
        out_type=jax.ShapeDtypeStruct((B, 2 * EMB), jnp.float32),
        mesh=mesh,
        scratch_types=[
            pltpu.VMEM((BPW * T_LEN,), jnp.int32),
            pltpu.VMEM((BPW * R_LEN + 8,), jnp.int32),
            pltpu.VMEM((BPW * (T_LEN // 4) + 16,), jnp.int32),
            pltpu.VMEM((BPW * (R_LEN // 4) + 16,), jnp.int32),
            pltpu.VMEM((RC0, 2 * EMB), jnp.float32),
            pltpu.VMEM((RC1, 2 * EMB), jnp.float32),
            pltpu.VMEM((RC0, 2 * EMB), jnp.float32),
            pltpu.VMEM((RC1, 2 * EMB), jnp.float32),
            pltpu.VMEM((BPW, 2 * EMB), jnp.float32),
            pltpu.SemaphoreType.DMA,
            pltpu.SemaphoreType.DMA,
        ],
    )(tg, rg, th8, rh8, emb2)


TRB = 32768
TRH = TRB // 2


def _tr_body(x_ref, o_ref):
    x = x_ref[...]
    for s in range(0, TRH, 512):
        o_ref[s:s + 512, 0:64] = jnp.transpose(x[:, s:s + 512])
        o_ref[s:s + 512, 64:128] = jnp.transpose(x[:, TRH + s:TRH + s + 512])


def _transpose(embT):
    nblk = (1000000 + TRB - 1) // TRB
    return pl.pallas_call(
        _tr_body,
        grid=(nblk,),
        in_specs=[pl.BlockSpec((64, TRB), lambda j: (0, j))],
        out_specs=pl.BlockSpec((TRH, 128), lambda j: (j, 0)),
        out_shape=jax.ShapeDtypeStruct((nblk * TRH, 128), jnp.float32),
    )(embT)


def _mlp_body(x_ref, w1_ref, b1_ref, w2_ref, b2_ref, w3_ref, b3_ref, o_ref):
    x = x_ref[...]
    h = jnp.dot(x, w1_ref[...], preferred_element_type=jnp.float32) + b1_ref[...]
    h = jnp.maximum(h, 0.0)
    h = jnp.dot(h, w2_ref[...], preferred_element_type=jnp.float32) + b2_ref[...]
    h = jnp.maximum(h, 0.0)
    o_ref[...] = jnp.dot(h, w3_ref[...], preferred_element_type=jnp.float32) + b3_ref[...]


def _mlp(x, w1t, b1r, w2t, b2r, w3t, b3r):
    BLK = 512
    return pl.pallas_call(
        _mlp_body,
        grid=(B // BLK,),
        in_specs=[
            pl.BlockSpec((BLK, 2 * EMB), lambda i: (i, 0)),
            pl.BlockSpec((2 * EMB, H1), lambda i: (0, 0)),
            pl.BlockSpec((1, H1), lambda i: (0, 0)),
            pl.BlockSpec((H1, H2), lambda i: (0, 0)),
            pl.BlockSpec((1, H2), lambda i: (0, 0)),
            pl.BlockSpec((H2, NCLS), lambda i: (0, 0)),
            pl.BlockSpec((1, NCLS), lambda i: (0, 0)),
        ],
        out_specs=pl.BlockSpec((BLK, NCLS), lambda i: (i, 0)),
        out_shape=jax.ShapeDtypeStruct((B, NCLS), jnp.float32),
    )(x, w1t, b1r, w2t, b2r, w3t, b3r)


def kernel(title_inputs, review_inputs, emb, W1, b1, W2, b2, W3, b3):
    tf = title_inputs.reshape(-1)
    rf = review_inputs.reshape(-1)
    tblk = jnp.right_shift(tf, 15)
    tloc = jnp.bitwise_and(tf, TRB - 1)
    rblk = jnp.right_shift(rf, 15)
    rloc = jnp.bitwise_and(rf, TRB - 1)
    tg = tblk * TRH + jnp.bitwise_and(tloc, TRH - 1)
    rg = rblk * TRH + jnp.bitwise_and(rloc, TRH - 1)
    th8 = jnp.left_shift(jnp.right_shift(tloc, 14), 6).astype(jnp.int8)
    rh8 = jnp.left_shift(jnp.right_shift(rloc, 14), 6).astype(jnp.int8)
    th32 = lax.bitcast_convert_type(th8.reshape(-1, 4), jnp.int32)
    rh32 = lax.bitcast_convert_type(rh8.reshape(-1, 4), jnp.int32)
    emb2 = _transpose(emb.T)
    combined = _bow(tg, rg, th32, rh32, emb2)
    return _mlp(combined, W1.T, b1.reshape(1, -1),
                W2.T, b2.reshape(1, -1), W3.T, b3.reshape(1, -1))
